# Initial kernel scaffold; baseline (speedup 1.0000x reference)
import functools

import jax
import jax.numpy as jnp
from jax import lax
from jax.experimental import pallas as pl
from jax.experimental.pallas import tpu as pltpu
from jax.experimental.pallas import tpu_sc as plsc

_NW = 32


@functools.partial(jax.jit, static_argnames=("n_chunks", "chunk"))
def _sc_gather(x_flat, table, *, n_chunks, chunk):
    n = x_flat.shape[0]
    d = table.shape[1]
    n_per_w = n // _NW
    mesh = plsc.VectorSubcoreMesh(core_axis_name="c", subcore_axis_name="s")

    @functools.partial(
        pl.kernel,
        mesh=mesh,
        out_type=jax.ShapeDtypeStruct((n, d), jnp.float32),
        scratch_types=[
            pltpu.VMEM((n_chunks, chunk), jnp.int32),
            pltpu.VMEM((chunk, d), jnp.float32),
            pltpu.SemaphoreType.DMA,
        ],
    )
    def k(idx_hbm, table_hbm, out_hbm, idx_v, rows_v, sem):
        wid = lax.axis_index("s") * 2 + lax.axis_index("c")
        base = wid * n_per_w
        for ch in range(n_chunks):
            pltpu.sync_copy(idx_hbm.at[pl.ds(base + ch * chunk, chunk)],
                            idx_v.at[ch])
        for ch in range(n_chunks):
            pltpu.async_copy(table_hbm.at[idx_v.at[ch]], rows_v, sem).wait()
            pltpu.sync_copy(rows_v, out_hbm.at[pl.ds(base + ch * chunk, chunk)])

    return k(x_flat, table)


def kernel(x, table):
    b, h = x.shape
    d = table.shape[1]
    n = b * h
    flat = x.reshape(n)
    out = _sc_gather(flat, table, n_chunks=16, chunk=n // _NW // 16)
    return out.reshape(b, h, d)


# speedup vs baseline: 1.1075x; 1.1075x over previous
import functools

import jax
import jax.numpy as jnp
from jax import lax
from jax.experimental import pallas as pl
from jax.experimental.pallas import tpu as pltpu
from jax.experimental.pallas import tpu_sc as plsc

_NW = 32


@functools.partial(jax.jit, static_argnames=("n_chunks", "chunk"))
def _sc_gather(x_flat, table, *, n_chunks, chunk):
    n = x_flat.shape[0]
    d = table.shape[1]
    n_per_w = n // _NW
    mesh = plsc.VectorSubcoreMesh(core_axis_name="c", subcore_axis_name="s")

    @functools.partial(
        pl.kernel,
        mesh=mesh,
        out_type=jax.ShapeDtypeStruct((n, d), jnp.float32),
        scratch_types=[
            pltpu.VMEM((n_per_w,), jnp.int32),
            pltpu.VMEM((chunk, d), jnp.float32),
            pltpu.SemaphoreType.DMA,
        ],
        compiler_params=pltpu.CompilerParams(use_tc_tiling_on_sc=False),
    )
    def k(idx_hbm, table_hbm, out_hbm, idx_v, rows_v, sem):
        wid = lax.axis_index("s") * 2 + lax.axis_index("c")
        base = wid * n_per_w
        pltpu.sync_copy(idx_hbm.at[pl.ds(base, n_per_w)], idx_v)
        for ch in range(n_chunks):
            pltpu.async_copy(table_hbm.at[idx_v.at[pl.ds(ch * chunk, chunk)]],
                             rows_v, sem).wait()
            pltpu.sync_copy(rows_v, out_hbm.at[pl.ds(base + ch * chunk, chunk)])

    return k(x_flat, table)


def kernel(x, table):
    b, h = x.shape
    d = table.shape[1]
    n = b * h
    flat = x.reshape(n)
    out = _sc_gather(flat, table, n_chunks=16, chunk=n // _NW // 16)
    return out.reshape(b, h, d)


# speedup vs baseline: 1.7375x; 1.5688x over previous
import functools

import jax
import jax.numpy as jnp
from jax import lax
from jax.experimental import pallas as pl
from jax.experimental.pallas import tpu as pltpu
from jax.experimental.pallas import tpu_sc as plsc

_NW = 32
_L = 16


@functools.partial(jax.jit, static_argnames=("n_chunks",))
def _sc_gather(x, table, *, n_chunks):
    b, h = x.shape
    d = table.shape[1]
    b_per_w = b // _NW
    b_chunk = b_per_w // n_chunks
    n_per_w = b_per_w * h
    chunk = b_chunk * h
    mesh = plsc.VectorSubcoreMesh(core_axis_name="c", subcore_axis_name="s")

    @functools.partial(
        pl.kernel,
        mesh=mesh,
        out_type=jax.ShapeDtypeStruct((b, h, d), jnp.float32),
        scratch_types=[
            pltpu.VMEM((b_per_w, h), jnp.int32),
            pltpu.VMEM((n_per_w,), jnp.int32),
            pltpu.VMEM((chunk, d), jnp.float32),
            pltpu.SemaphoreType.DMA,
            pltpu.SemaphoreType.DMA,
        ],
        compiler_params=pltpu.CompilerParams(use_tc_tiling_on_sc=False),
    )
    def k(x_hbm, table_hbm, out_hbm, idx2d_v, idx_v, rows_v, sem, sem_s):
        wid = lax.axis_index("s") * 2 + lax.axis_index("c")
        base_b = wid * b_per_w
        pltpu.sync_copy(x_hbm.at[pl.ds(base_b, b_per_w), :], idx2d_v)

        col_offs = [c * _L for c in range(h // _L)] + [h - _L]

        def flatten_step(r, _):
            for c in col_offs:
                idx_v[pl.ds(r * h + c, _L)] = idx2d_v[r, pl.ds(c, _L)]
            return 0

        lax.fori_loop(0, b_per_w, flatten_step, 0)

        def store_step(bb, ch):
            pltpu.sync_copy(rows_v.at[pl.ds(bb * h, h), :],
                            out_hbm.at[base_b + ch * b_chunk + bb])
            return ch

        for ch in range(n_chunks):
            idx_ref = idx_v.at[pl.ds(ch * chunk, chunk)]
            pltpu.async_copy(table_hbm.at[idx_ref], rows_v, sem).wait()
            lax.fori_loop(0, b_chunk, store_step, ch)

    return k(x, table)


def kernel(x, table):
    return _sc_gather(x, table, n_chunks=16)
